# baseline (speedup 1.0000x reference)
import jax
import jax.numpy as jnp
from jax.experimental import pallas as pl

N = 10000
NG = 64
EPS = 1e-5


def _pdn_conv(x, src, dst, edge_attr, W, w1, b1, w2, b2):
    h = jax.nn.relu(edge_attr @ w1 + b1)
    ew = jax.nn.sigmoid(h @ w2 + b2)[:, 0]
    n = x.shape[0]
    loops = jnp.arange(n, dtype=src.dtype)
    s = jnp.concatenate([src, loops])
    d = jnp.concatenate([dst, loops])
    ew = jnp.concatenate([ew, jnp.ones((n,), dtype=ew.dtype)])
    deg = jnp.zeros((n,), dtype=ew.dtype).at[d].add(ew)
    dinv = jnp.where(deg > 0, jax.lax.rsqrt(jnp.maximum(deg, 1e-12)), 0.0)
    norm = dinv[s] * ew * dinv[d]
    xw = x @ W
    msg = norm[:, None] * jnp.take(xw, s, axis=0)
    return jnp.zeros_like(xw).at[d].add(msg)


def _bn(x):
    mu = jnp.mean(x, axis=0, keepdims=True)
    var = jnp.mean((x - mu) ** 2, axis=0, keepdims=True)
    return (x - mu) * jax.lax.rsqrt(var + EPS)


def _fc_kernel(p_ref, w_ref, b_ref, o_ref):
    o_ref[...] = p_ref[...] @ w_ref[...] + b_ref[...]


def kernel(x, edge_index, batch, dropout, edge_attr, device, Wlin, mW1, mb1, mW2, mb2, Wfc, bfc):
    src, dst = edge_index[0], edge_index[1]
    h = _pdn_conv(x, src, dst, edge_attr, Wlin[0], mW1[0], mb1[0], mW2[0], mb2[0])
    h0 = h
    for i in range(2):
        h = jax.nn.relu(_bn(h))
        h = _pdn_conv(h, src, dst, edge_attr, Wlin[1 + i], mW1[1 + i], mb1[1 + i], mW2[1 + i], mb2[1 + i])
    h1 = h + h0
    h = h + h0
    for i in range(2):
        h = jax.nn.relu(_bn(h))
        h = _pdn_conv(h, src, dst, edge_attr, Wlin[3 + i], mW1[3 + i], mb1[3 + i], mW2[3 + i], mb2[3 + i])
    h2 = jax.nn.relu(h + h0 + h1)
    pooled = jax.ops.segment_max(h2, batch, num_segments=NG)
    pooled = jnp.where(jnp.isfinite(pooled), pooled, 0.0)
    out = pl.pallas_call(
        _fc_kernel,
        out_shape=jax.ShapeDtypeStruct((NG, Wfc.shape[1]), jnp.float32),
    )(pooled, Wfc, bfc[None, :])
    return out


# speedup vs baseline: 7.8047x; 7.8047x over previous
import functools

import jax
import jax.numpy as jnp
from jax import lax
from jax.experimental import pallas as pl
from jax.experimental.pallas import tpu as pltpu
from jax.experimental.pallas import tpu_sc as plsc

N = 10000
E = 320000
D = 128
DE = 16
NC = 40
NG = 64
NL = 5
EPS = 1e-5

NCORE = 2
NSUB = 16
NW = NCORE * NSUB
EPW = E // NW
K = 80
NCHUNK = EPW // K
NPAD = 10240
RPS = NPAD // NSUB

_MESH = plsc.VectorSubcoreMesh(core_axis_name="c", subcore_axis_name="s")


def _edge_mlp_body(ea_ref, w1_ref, b1_ref, w2_ref, b2_ref, *ewc_refs):
    ea = ea_ref[...]
    for l in range(NL):
        h = jnp.maximum(jnp.dot(ea, w1_ref[l], preferred_element_type=jnp.float32)
                        + b1_ref[l][None, :], 0.0)
        z = jnp.dot(h, w2_ref[l], preferred_element_type=jnp.float32) + b2_ref[l][None, :]
        ewc_refs[l][0, 0, :] = jax.nn.sigmoid(z)[:, 0]


def _edge_mlp(edge_attr, mW1, mb1, mW2, mb2):
    BE = 8000
    return pl.pallas_call(
        _edge_mlp_body,
        grid=(E // BE,),
        in_specs=[
            pl.BlockSpec((BE, DE), lambda i: (i, 0)),
            pl.BlockSpec((NL, DE, DE), lambda i: (0, 0, 0)),
            pl.BlockSpec((NL, DE), lambda i: (0, 0)),
            pl.BlockSpec((NL, DE, 1), lambda i: (0, 0, 0)),
            pl.BlockSpec((NL, 1), lambda i: (0, 0)),
        ],
        out_specs=[pl.BlockSpec((1, 1, BE), lambda i: (i, 0, 0)) for _ in range(NL)],
        out_shape=[jax.ShapeDtypeStruct((E // BE, 1, BE), jnp.float32) for _ in range(NL)],
    )(edge_attr, mW1, mb1, mW2, mb2)


def _sc_deg_body(dst_hbm, e0, e1, e2, e3, e4, z_hbm, out_hbm,
                 dst_v, w0, w1, w2, w3, w4, rows, acc_sh):
    c = lax.axis_index("c")
    s = lax.axis_index("s")
    wid = s * NCORE + c
    ews = (e0, e1, e2, e3, e4)
    ewv = (w0, w1, w2, w3, w4)

    pltpu.sync_copy(z_hbm.at[pl.ds(s * RPS, RPS), :],
                    acc_sh.at[pl.ds(s * RPS, RPS), :])

    def zrow2(i, _):
        for cc in range(NL, D // 16):
            rows[i, pl.ds(cc * 16, 16)] = jnp.zeros((16,), jnp.float32)
        return 0

    lax.fori_loop(0, K, zrow2, 0)
    plsc.subcore_barrier()

    def chunk(j, _):
        base = wid * EPW + j * K
        pltpu.sync_copy(dst_hbm.at[pl.ds(base, K)], dst_v)
        for l in range(NL):
            pltpu.sync_copy(ews[l].at[pl.ds(base, K)], ewv[l].at[pl.ds(0, K)])

        def row(i, _):
            for l in range(NL):
                ev = ewv[l][pl.ds(i, 16)]
                rows[i, pl.ds(l * 16, 16)] = jnp.full((16,), ev[0], jnp.float32)
            return 0

        lax.fori_loop(0, K, row, 0)
        pltpu.sync_copy(rows, acc_sh.at[dst_v], add=True)
        return 0

    lax.fori_loop(0, NCHUNK, chunk, 0)
    plsc.subcore_barrier()
    pltpu.sync_copy(acc_sh.at[pl.ds(s * RPS, RPS), :],
                    out_hbm.at[c, pl.ds(s * RPS, RPS), :])


def _sc_deg(dst, ewc):
    z = jnp.zeros((NPAD, D), jnp.float32)
    return pl.kernel(
        _sc_deg_body,
        out_type=jax.ShapeDtypeStruct((NCORE, NPAD, D), jnp.float32),
        mesh=_MESH,
        scratch_types=[pltpu.VMEM((K,), jnp.int32)]
        + [pltpu.VMEM((K + 16,), jnp.float32) for _ in range(NL)]
        + [pltpu.VMEM((K, D), jnp.float32),
           pltpu.VMEM_SHARED((NPAD, D), jnp.float32)],
    )(dst, *ewc, z)


def _sc_agg_body(src_hbm, dst_hbm, ewc_hbm, y_hbm, z_hbm, out_hbm,
                 src_v, dst_v, ew_v, rows, acc_sh, sem):
    c = lax.axis_index("c")
    s = lax.axis_index("s")
    wid = s * NCORE + c

    pltpu.sync_copy(z_hbm.at[pl.ds(s * RPS, RPS), :],
                    acc_sh.at[pl.ds(s * RPS, RPS), :])
    plsc.subcore_barrier()

    def chunk(j, _):
        base = wid * EPW + j * K
        pltpu.sync_copy(src_hbm.at[pl.ds(base, K)], src_v)
        pltpu.sync_copy(dst_hbm.at[pl.ds(base, K)], dst_v)
        pltpu.sync_copy(ewc_hbm.at[pl.ds(base, K)], ew_v.at[pl.ds(0, K)])
        pltpu.async_copy(y_hbm.at[src_v], rows, sem).wait()

        def row(i, _):
            ev = ew_v[pl.ds(i, 16)]
            sv = jnp.full((16,), ev[0], jnp.float32)
            for cc in range(D // 16):
                rows[i, pl.ds(cc * 16, 16)] = rows[i, pl.ds(cc * 16, 16)] * sv
            return 0

        lax.fori_loop(0, K, row, 0)
        pltpu.sync_copy(rows, acc_sh.at[dst_v], add=True)
        return 0

    lax.fori_loop(0, NCHUNK, chunk, 0)
    plsc.subcore_barrier()
    pltpu.sync_copy(acc_sh.at[pl.ds(s * RPS, RPS), :],
                    out_hbm.at[c, pl.ds(s * RPS, RPS), :])


def _sc_agg(src, dst, ewc_l, y, z):
    return pl.kernel(
        _sc_agg_body,
        out_type=jax.ShapeDtypeStruct((NCORE, NPAD, D), jnp.float32),
        mesh=_MESH,
        scratch_types=[
            pltpu.VMEM((K,), jnp.int32),
            pltpu.VMEM((K,), jnp.int32),
            pltpu.VMEM((K + 16,), jnp.float32),
            pltpu.VMEM((K, D), jnp.float32),
            pltpu.VMEM_SHARED((NPAD, D), jnp.float32),
            pltpu.SemaphoreType.DMA,
        ],
    )(src, dst, ewc_l, y, z)


def _dinv_body(p_ref, o_ref):
    o_ref[...] = lax.rsqrt(1.0 + p_ref[0] + p_ref[1])


def _dinv(degp):
    return pl.pallas_call(
        _dinv_body,
        out_shape=jax.ShapeDtypeStruct((NPAD, D), jnp.float32),
    )(degp)


BN_ROWS = 2000
NBLK = N // BN_ROWS


def _y0_body(lidx, x_ref, w_ref, dv_ref, y_ref):
    xw = jnp.dot(x_ref[...], w_ref[...], preferred_element_type=jnp.float32)
    y_ref[...] = xw * dv_ref[:, 16 * lidx:16 * lidx + 1]


def _y_bn_body(lidx, s_ref, st_ref, w_ref, dv_ref, y_ref):
    mu = st_ref[0] / N
    var = st_ref[1] / N - mu * mu
    h = jnp.maximum((s_ref[...] - mu[None, :]) * lax.rsqrt(var + EPS)[None, :], 0.0)
    xw = jnp.dot(h, w_ref[...], preferred_element_type=jnp.float32)
    y_ref[...] = xw * dv_ref[:, 16 * lidx:16 * lidx + 1]


_DV_SPEC = pl.BlockSpec((BN_ROWS, D), lambda i: (i, 0))


def _y0(lidx, x, W, dv):
    return pl.pallas_call(
        functools.partial(_y0_body, lidx),
        grid=(NBLK,),
        in_specs=[
            pl.BlockSpec((BN_ROWS, D), lambda i: (i, 0)),
            pl.BlockSpec((D, D), lambda i: (0, 0)),
            _DV_SPEC,
        ],
        out_specs=pl.BlockSpec((BN_ROWS, D), lambda i: (i, 0)),
        out_shape=jax.ShapeDtypeStruct((N, D), jnp.float32),
    )(x, W, dv)


def _y_bn(lidx, s_prev, stats, W, dv):
    return pl.pallas_call(
        functools.partial(_y_bn_body, lidx),
        grid=(NBLK,),
        in_specs=[
            pl.BlockSpec((BN_ROWS, D), lambda i: (i, 0)),
            pl.BlockSpec((2, D), lambda i: (0, 0)),
            pl.BlockSpec((D, D), lambda i: (0, 0)),
            _DV_SPEC,
        ],
        out_specs=pl.BlockSpec((BN_ROWS, D), lambda i: (i, 0)),
        out_shape=jax.ShapeDtypeStruct((N, D), jnp.float32),
    )(s_prev, stats, W, dv)


def _comb_body(lidx, nskip, want_stats, p_ref, y_ref, dv_ref, *rest):
    skips = rest[:nskip]
    if want_stats:
        s_ref, st_ref, acc = rest[nskip], rest[nskip + 1], rest[nskip + 2]
    else:
        s_ref = rest[nskip]
    o = (p_ref[0] + p_ref[1] + y_ref[...]) * dv_ref[:, 16 * lidx:16 * lidx + 1]
    for sk in skips:
        o = o + sk[...]
    s_ref[...] = o
    if want_stats:
        i = pl.program_id(0)

        @pl.when(i == 0)
        def _():
            acc[...] = jnp.zeros_like(acc)

        acc[0, :] += jnp.sum(o, axis=0)
        acc[1, :] += jnp.sum(o * o, axis=0)

        @pl.when(i == NBLK - 1)
        def _():
            st_ref[...] = acc[...]


def _combine(lidx, P, y, dv, skips, want_stats):
    nskip = len(skips)
    blk = pl.BlockSpec((BN_ROWS, D), lambda i: (i, 0))
    in_specs = [pl.BlockSpec((2, BN_ROWS, D), lambda i: (0, i, 0)), blk,
                _DV_SPEC] + [blk] * nskip
    out_specs = [blk]
    out_shape = [jax.ShapeDtypeStruct((N, D), jnp.float32)]
    scratch = []
    if want_stats:
        out_specs.append(pl.BlockSpec((2, D), lambda i: (0, 0)))
        out_shape.append(jax.ShapeDtypeStruct((2, D), jnp.float32))
        scratch.append(pltpu.VMEM((2, D), jnp.float32))
    res = pl.pallas_call(
        functools.partial(_comb_body, lidx, nskip, want_stats),
        grid=(NBLK,),
        in_specs=in_specs,
        out_specs=out_specs,
        out_shape=out_shape,
        scratch_shapes=scratch,
    )(P, y, dv, *skips)
    return res if want_stats else (res[0], None)


def _pool_body(s4_ref, b_ref, wfc_ref, bfc_ref, o_ref, acc):
    i = pl.program_id(0)

    @pl.when(i == 0)
    def _():
        acc[...] = jnp.full_like(acc, -jnp.inf)

    h2 = jnp.maximum(s4_ref[...], 0.0)
    b = b_ref[...]
    neg = jnp.float32(-jnp.inf)
    for g in range(NG):
        m = (b == g)
        mg = jnp.max(jnp.where(m, h2, neg), axis=0)
        acc[g, :] = jnp.maximum(acc[g, :], mg)

    @pl.when(i == NBLK - 1)
    def _():
        pooled = jnp.where(jnp.isfinite(acc[...]), acc[...], 0.0)
        o_ref[...] = (jnp.dot(pooled, wfc_ref[...],
                              preferred_element_type=jnp.float32)
                      + bfc_ref[0, :][None, :])


def _pool_fc(s4, batch2, Wfc, bfc):
    return pl.pallas_call(
        _pool_body,
        grid=(NBLK,),
        in_specs=[
            pl.BlockSpec((BN_ROWS, D), lambda i: (i, 0)),
            pl.BlockSpec((BN_ROWS, 1), lambda i: (i, 0)),
            pl.BlockSpec((D, NC), lambda i: (0, 0)),
            pl.BlockSpec((1, NC), lambda i: (0, 0)),
        ],
        out_specs=pl.BlockSpec((NG, NC), lambda i: (0, 0)),
        out_shape=jax.ShapeDtypeStruct((NG, NC), jnp.float32),
        scratch_shapes=[pltpu.VMEM((NG, D), jnp.float32)],
    )(s4, batch2, Wfc, bfc)


def kernel(x, edge_index, batch, dropout, edge_attr, device,
           Wlin, mW1, mb1, mW2, mb2, Wfc, bfc):
    src_ = edge_index[0]
    dst = edge_index[1]
    batch2 = batch.reshape(N, 1)

    ewc = _edge_mlp(edge_attr, mW1, mb1, mW2, mb2)
    ewc = [w.reshape(E) for w in ewc]
    degp = _sc_deg(dst, ewc)
    dinv = _dinv(degp)

    y = _y0(0, x, Wlin[0], dinv)
    zzz = jnp.zeros((NPAD, D), jnp.float32)
    P = _sc_agg(src_, dst, ewc[0], y, zzz)
    s0, st = _combine(0, P, y, dinv, [], True)
    y = _y_bn(1, s0, st, Wlin[1], dinv)
    P = _sc_agg(src_, dst, ewc[1], y, zzz)
    s1, st = _combine(1, P, y, dinv, [], True)
    y = _y_bn(2, s1, st, Wlin[2], dinv)
    P = _sc_agg(src_, dst, ewc[2], y, zzz)
    s2, st = _combine(2, P, y, dinv, [s0], True)
    y = _y_bn(3, s2, st, Wlin[3], dinv)
    P = _sc_agg(src_, dst, ewc[3], y, zzz)
    s3, st = _combine(3, P, y, dinv, [], True)
    y = _y_bn(4, s3, st, Wlin[4], dinv)
    P = _sc_agg(src_, dst, ewc[4], y, zzz)
    s4, _ = _combine(4, P, y, dinv, [s0, s2], False)

    return _pool_fc(s4, batch2, Wfc, bfc.reshape(1, NC))
